# baseline (speedup 1.0000x reference)
import functools
import jax
import jax.numpy as jnp
from jax import lax
from jax.experimental import pallas as pl
from jax.experimental.pallas import tpu as pltpu
from jax.experimental.pallas import tpu_sc as plsc

_D = 768
_NV = _D // 16
_CHUNK = 32


def _rsqrt_newton(v):
    i = lax.bitcast_convert_type(v, jnp.int32)
    i = jnp.int32(0x5F3759DF) - lax.shift_right_logical(i, 1)
    y = lax.bitcast_convert_type(i, jnp.float32)
    for _ in range(3):
        y = y * (1.5 - 0.5 * v * y * y)
    return y


def _sc_body(S, rows_per_w, x_hbm, p_hbm, g_hbm, b_hbm, o_hbm,
             xc, pc, oc, gv, bv, sv):
    nc = 2
    wid = lax.axis_index("s") * nc + lax.axis_index("c")
    base = wid * rows_per_w
    pbase = lax.rem(base, S)

    pltpu.sync_copy(g_hbm, gv)
    pltpu.sync_copy(b_hbm, bv)

    def chunk_body(c, _):
        rb = base + c * _CHUNK
        pb = pbase + c * _CHUNK
        pltpu.sync_copy(x_hbm.at[pl.ds(rb, _CHUNK)], xc)
        pltpu.sync_copy(p_hbm.at[pl.ds(pb, _CHUNK)], pc)

        def row_body(j, _):
            acc = jnp.zeros((16,), jnp.float32)
            acc2 = jnp.zeros((16,), jnp.float32)
            for k in range(_NV):
                v = xc[j, pl.ds(16 * k, 16)] + pc[j, pl.ds(16 * k, 16)]
                oc[j, pl.ds(16 * k, 16)] = v
                acc = acc + v
                acc2 = acc2 + v * v
            s1 = acc[0]
            s2 = acc2[0]
            for t in range(1, 16):
                s1 = s1 + acc[t]
                s2 = s2 + acc2[t]
            mean = s1 * (1.0 / _D)
            var = s2 * (1.0 / _D) - mean * mean
            inv = _rsqrt_newton(var + 1e-5)
            for k in range(_NV):
                v = oc[j, pl.ds(16 * k, 16)]
                oc[j, pl.ds(16 * k, 16)] = (
                    (v - mean) * inv * gv[pl.ds(16 * k, 16)]
                    + bv[pl.ds(16 * k, 16)]
                )
            return 0

        lax.fori_loop(0, _CHUNK, row_body, 0)
        pltpu.sync_copy(oc, o_hbm.at[pl.ds(rb, _CHUNK)])
        return 0

    lax.fori_loop(0, rows_per_w // _CHUNK, chunk_body, 0)


def kernel(x, pos_table, ln_gamma, ln_beta):
    B, S, D = x.shape
    rows = B * S
    nw = 32
    rows_per_w = rows // nw
    x2 = x.reshape(rows, D)
    mesh = plsc.VectorSubcoreMesh(core_axis_name="c", subcore_axis_name="s")
    k = pl.kernel(
        functools.partial(_sc_body, S, rows_per_w),
        out_type=jax.ShapeDtypeStruct((rows, D), jnp.float32),
        mesh=mesh,
        scratch_types=[
            pltpu.VMEM((_CHUNK, D), jnp.float32),
            pltpu.VMEM((_CHUNK, D), jnp.float32),
            pltpu.VMEM((_CHUNK, D), jnp.float32),
            pltpu.VMEM((D,), jnp.float32),
            pltpu.VMEM((D,), jnp.float32),
            pltpu.VMEM((32,), jnp.float32),
        ],
    )
    out = k(x2, pos_table, ln_gamma, ln_beta)
    return out.reshape(B, S, D)


# speedup vs baseline: 7.3081x; 7.3081x over previous
import jax
import jax.numpy as jnp
from jax.experimental import pallas as pl
from jax.experimental.pallas import tpu as pltpu

_BS = 2048


def _ln_body(x_ref, p_ref, g_ref, b_ref, o_ref, *, n_pos_blocks):
    i = pl.program_id(0)
    s = jax.lax.rem(i, n_pos_blocks)
    emb = x_ref[...] + p_ref[pl.ds(s * _BS, _BS), :]
    mean = jnp.mean(emb, axis=-1, keepdims=True)
    d = emb - mean
    var = jnp.mean(d * d, axis=-1, keepdims=True)
    o_ref[...] = d * jax.lax.rsqrt(var + 1e-5) * g_ref[...] + b_ref[...]


def kernel(x, pos_table, ln_gamma, ln_beta):
    import functools
    B, S, D = x.shape
    rows = B * S
    x2 = x.reshape(rows, D)
    g2 = ln_gamma.reshape(1, D)
    b2 = ln_beta.reshape(1, D)
    n_pos_blocks = S // _BS

    out = pl.pallas_call(
        functools.partial(_ln_body, n_pos_blocks=n_pos_blocks),
        grid=(rows // _BS,),
        in_specs=[
            pl.BlockSpec((_BS, D), lambda i: (i, 0)),
            pl.BlockSpec((S, D), lambda i: (0, 0)),
            pl.BlockSpec((1, D), lambda i: (0, 0)),
            pl.BlockSpec((1, D), lambda i: (0, 0)),
        ],
        out_specs=pl.BlockSpec((_BS, D), lambda i: (i, 0)),
        out_shape=jax.ShapeDtypeStruct((rows, D), x.dtype),
        compiler_params=pltpu.CompilerParams(
            dimension_semantics=("parallel",),
        ),
    )(x2, pos_table, g2, b2)
    return out.reshape(B, S, D)
